# baseline (speedup 1.0000x reference)
import functools

import jax
import jax.numpy as jnp
from jax import lax
from jax.experimental import pallas as pl
from jax.experimental.pallas import tpu as pltpu
from jax.experimental.pallas import tpu_sc as plsc

B, S, H = 4, 2048, 1024
T = B * S
OFFSET = 2

NSEG = 1
TSEG = T // NSEG

NW = 32
ROWS_PER_TILE = TSEG // NW
CHUNK = 32
NCHUNK = ROWS_PER_TILE // CHUNK


def _sc_gather(word_emb, ids2d):
  mesh = plsc.VectorSubcoreMesh(core_axis_name="c", subcore_axis_name="s")

  @functools.partial(
      pl.kernel,
      mesh=mesh,
      out_type=jax.ShapeDtypeStruct((TSEG, H), jnp.float32),
      scratch_types=[
          pltpu.VMEM((NCHUNK, CHUNK), jnp.int32),
          pltpu.VMEM((CHUNK, H), jnp.float32),
          pltpu.VMEM((CHUNK, H), jnp.float32),
          pltpu.VMEM((CHUNK, H), jnp.float32),
          pltpu.SemaphoreType.DMA,
          pltpu.SemaphoreType.DMA,
          pltpu.SemaphoreType.DMA,
          pltpu.SemaphoreType.DMA,
          pltpu.SemaphoreType.DMA,
          pltpu.SemaphoreType.DMA,
      ],
  )
  def gather_kernel(table, idx, out, idx_v, buf0, buf1, buf2,
                    g0, g1, g2, o0, o1, o2):
    wid = lax.axis_index("s") * 2 + lax.axis_index("c")
    chunk0 = wid * NCHUNK
    pltpu.sync_copy(idx.at[pl.ds(chunk0, NCHUNK)], idx_v)
    bufs = (buf0, buf1, buf2)
    gsems = (g0, g1, g2)
    osems = (o0, o1, o2)
    nbuf = len(bufs)
    gcp = [None] * nbuf
    ocp = [None] * nbuf
    for c in range(min(nbuf - 1, NCHUNK)):
      gcp[c] = pltpu.async_copy(table.at[idx_v.at[c]], bufs[c], gsems[c])
    for c in range(NCHUNK):
      b = c % nbuf
      pc = c + nbuf - 1
      if pc < NCHUNK:
        pb = pc % nbuf
        if ocp[pb] is not None:
          ocp[pb].wait()
        gcp[pb] = pltpu.async_copy(table.at[idx_v.at[pc]], bufs[pb], gsems[pb])
      gcp[b].wait()
      row0 = (chunk0 + c) * CHUNK
      ocp[b] = pltpu.async_copy(bufs[b], out.at[pl.ds(row0, CHUNK)], osems[b])
    for b in range(min(nbuf, NCHUNK)):
      ocp[b].wait()

  return gather_kernel(word_emb, ids2d)


TOK = 2048


HH = H // 2


def _tc_embed_ln_body(ga_ref, gb_ref, pa_ref, pb_ref, tt_id_ref, ut_id_ref,
                      tt_ref, ut_ref, gamma_ref, beta_ref, out_ref):
  t = tt_id_ref[0]
  u = ut_id_ref[0]
  oh_t = (lax.broadcasted_iota(jnp.int32, (2, TOK), 0) == t).astype(jnp.float32)
  oh_u = (lax.broadcasted_iota(jnp.int32, (8, TOK), 0) == u).astype(jnp.float32)
  dn = (((0,), (0,)), ((), ()))
  halves = []
  for hs, g_ref, p_ref in ((0, ga_ref, pa_ref), (1, gb_ref, pb_ref)):
    col = pl.ds(hs * HH, HH)
    tt_c = lax.dot_general(oh_t, tt_ref[:, col], dn,
                           preferred_element_type=jnp.float32,
                           precision=lax.Precision.DEFAULT)
    ut_c = lax.dot_general(oh_u, ut_ref[:, col], dn,
                           preferred_element_type=jnp.float32,
                           precision=lax.Precision.DEFAULT)
    halves.append(g_ref[...] + p_ref[...] + tt_c + ut_c)
  xa, xb = halves
  mu = (jnp.sum(xa, -1, keepdims=True) + jnp.sum(xb, -1, keepdims=True)) \
      * jnp.float32(1.0 / H)
  xca = xa - mu
  xcb = xb - mu
  var = (jnp.sum(xca * xca, -1, keepdims=True)
         + jnp.sum(xcb * xcb, -1, keepdims=True)) * jnp.float32(1.0 / H)
  r = lax.rsqrt(var + 1e-5)
  out_ref[:, pl.ds(0, HH)] = xca * r * gamma_ref[:, pl.ds(0, HH)] \
      + beta_ref[:, pl.ds(0, HH)]
  out_ref[:, pl.ds(HH, HH)] = xcb * r * gamma_ref[:, pl.ds(HH, HH)] \
      + beta_ref[:, pl.ds(HH, HH)]


def _tc_embed_ln(seg, prev_out, gath, pos_used, tt_ids, ut_ids, tt_emb, ut_emb,
                 gamma2, beta2):
  nsh = S // TOK
  nbh = B // NSEG
  base = seg * nbh
  body = _tc_embed_ln_body
  if prev_out is not None:
    body = lambda prev_ref, *refs: _tc_embed_ln_body(*refs)
  in_specs = [
      pl.BlockSpec((TOK, HH), lambda sh, b: (b * nsh + sh, 0)),
      pl.BlockSpec((TOK, HH), lambda sh, b: (b * nsh + sh, 1)),
      pl.BlockSpec((TOK, HH), lambda sh, b: (sh, 0)),
      pl.BlockSpec((TOK, HH), lambda sh, b: (sh, 1)),
      pl.BlockSpec((1, 1, TOK), lambda sh, b, base=base, nsh=nsh:
                   ((base + b) * nsh + sh, 0, 0)),
      pl.BlockSpec((1, 1, TOK), lambda sh, b, base=base, nsh=nsh:
                   ((base + b) * nsh + sh, 0, 0)),
      pl.BlockSpec((2, H), lambda sh, b: (0, 0)),
      pl.BlockSpec((8, H), lambda sh, b: (0, 0)),
      pl.BlockSpec((1, H), lambda sh, b: (0, 0)),
      pl.BlockSpec((1, H), lambda sh, b: (0, 0)),
  ]
  args = (gath, gath, pos_used, pos_used, tt_ids, ut_ids, tt_emb, ut_emb,
          gamma2, beta2)
  aliases = {}
  if prev_out is not None:
    in_specs = [pl.BlockSpec(memory_space=pl.ANY)] + in_specs
    args = (prev_out,) + args
    aliases = {0: 0}
  return pl.pallas_call(
      body,
      grid=(nsh, nbh),
      in_specs=in_specs,
      out_specs=pl.BlockSpec((TOK, H), lambda sh, b, base=base, nsh=nsh:
                             ((base + b) * nsh + sh, 0)),
      out_shape=jax.ShapeDtypeStruct((T, H), jnp.float32),
      input_output_aliases=aliases,
  )(*args)


def kernel(input_ids, token_type_ids, user_type_ids, word_emb, pos_emb, tt_emb,
           ut_emb, gamma, beta):
  pos_used = lax.slice(pos_emb, (OFFSET, 0), (OFFSET + S, H))
  tt_ids = token_type_ids.reshape(T // TOK, 1, TOK).astype(jnp.int32)
  ut_ids = user_type_ids.reshape(T // TOK, 1, TOK).astype(jnp.int32)
  gamma2 = gamma.reshape(1, H)
  beta2 = beta.reshape(1, H)
  ids_seg = input_ids.reshape(NSEG, NW * NCHUNK, CHUNK).astype(jnp.int32)

  gaths = [_sc_gather(word_emb, ids_seg[s]) for s in range(NSEG)]
  out = None
  for s in range(NSEG):
    out = _tc_embed_ln(s, out, gaths[s], pos_used, tt_ids, ut_ids,
                       tt_emb, ut_emb, gamma2, beta2)
  return out.reshape(B, S, H)


# speedup vs baseline: 1.0010x; 1.0010x over previous
import functools

import jax
import jax.numpy as jnp
from jax import lax
from jax.experimental import pallas as pl
from jax.experimental.pallas import tpu as pltpu
from jax.experimental.pallas import tpu_sc as plsc

B, S, H = 4, 2048, 1024
T = B * S
OFFSET = 2

NSEG = 1
TSEG = T // NSEG

NW = 32
ROWS_PER_TILE = TSEG // NW
CHUNK = 32
NCHUNK = ROWS_PER_TILE // CHUNK


def _sc_gather(word_emb, ids2d):
  mesh = plsc.VectorSubcoreMesh(core_axis_name="c", subcore_axis_name="s")

  @functools.partial(
      pl.kernel,
      mesh=mesh,
      out_type=jax.ShapeDtypeStruct((TSEG, H), jnp.float32),
      scratch_types=[
          pltpu.VMEM((NCHUNK, CHUNK), jnp.int32),
          pltpu.VMEM((CHUNK, H), jnp.float32),
          pltpu.VMEM((CHUNK, H), jnp.float32),
          pltpu.VMEM((CHUNK, H), jnp.float32),
          pltpu.SemaphoreType.DMA,
          pltpu.SemaphoreType.DMA,
          pltpu.SemaphoreType.DMA,
          pltpu.SemaphoreType.DMA,
          pltpu.SemaphoreType.DMA,
          pltpu.SemaphoreType.DMA,
      ],
  )
  def gather_kernel(table, idx, out, idx_v, buf0, buf1, buf2,
                    g0, g1, g2, o0, o1, o2):
    wid = lax.axis_index("s") * 2 + lax.axis_index("c")
    chunk0 = wid * NCHUNK
    pltpu.sync_copy(idx.at[pl.ds(chunk0, NCHUNK)], idx_v)
    bufs = (buf0, buf1, buf2)
    gsems = (g0, g1, g2)
    osems = (o0, o1, o2)
    nbuf = len(bufs)
    gcp = [None] * nbuf
    ocp = [None] * nbuf
    for c in range(min(nbuf - 1, NCHUNK)):
      gcp[c] = pltpu.async_copy(table.at[idx_v.at[c]], bufs[c], gsems[c])
    for c in range(NCHUNK):
      b = c % nbuf
      pc = c + nbuf - 1
      if pc < NCHUNK:
        pb = pc % nbuf
        if ocp[pb] is not None:
          ocp[pb].wait()
        gcp[pb] = pltpu.async_copy(table.at[idx_v.at[pc]], bufs[pb], gsems[pb])
      gcp[b].wait()
      row0 = (chunk0 + c) * CHUNK
      ocp[b] = pltpu.async_copy(bufs[b], out.at[pl.ds(row0, CHUNK)], osems[b])
    for b in range(min(nbuf, NCHUNK)):
      ocp[b].wait()

  return gather_kernel(word_emb, ids2d)


TOK = 1024


HH = H // 2


def _tc_embed_ln_body(ga_ref, gb_ref, pa_ref, pb_ref, tt_id_ref, ut_id_ref,
                      tt_ref, ut_ref, gamma_ref, beta_ref, out_ref):
  t = tt_id_ref[0]
  u = ut_id_ref[0]
  oh_t = (lax.broadcasted_iota(jnp.int32, (2, TOK), 0) == t).astype(jnp.float32)
  oh_u = (lax.broadcasted_iota(jnp.int32, (8, TOK), 0) == u).astype(jnp.float32)
  dn = (((0,), (0,)), ((), ()))
  halves = []
  for hs, g_ref, p_ref in ((0, ga_ref, pa_ref), (1, gb_ref, pb_ref)):
    col = pl.ds(hs * HH, HH)
    tt_c = lax.dot_general(oh_t, tt_ref[:, col], dn,
                           preferred_element_type=jnp.float32,
                           precision=lax.Precision.DEFAULT)
    ut_c = lax.dot_general(oh_u, ut_ref[:, col], dn,
                           preferred_element_type=jnp.float32,
                           precision=lax.Precision.DEFAULT)
    halves.append(g_ref[...] + p_ref[...] + tt_c + ut_c)
  xa, xb = halves
  mu = (jnp.sum(xa, -1, keepdims=True) + jnp.sum(xb, -1, keepdims=True)) \
      * jnp.float32(1.0 / H)
  xca = xa - mu
  xcb = xb - mu
  var = (jnp.sum(xca * xca, -1, keepdims=True)
         + jnp.sum(xcb * xcb, -1, keepdims=True)) * jnp.float32(1.0 / H)
  r = lax.rsqrt(var + 1e-5)
  out_ref[:, pl.ds(0, HH)] = xca * r * gamma_ref[:, pl.ds(0, HH)] \
      + beta_ref[:, pl.ds(0, HH)]
  out_ref[:, pl.ds(HH, HH)] = xcb * r * gamma_ref[:, pl.ds(HH, HH)] \
      + beta_ref[:, pl.ds(HH, HH)]


def _tc_embed_ln(seg, prev_out, gath, pos_used, tt_ids, ut_ids, tt_emb, ut_emb,
                 gamma2, beta2):
  nsh = S // TOK
  nbh = B // NSEG
  base = seg * nbh
  body = _tc_embed_ln_body
  if prev_out is not None:
    body = lambda prev_ref, *refs: _tc_embed_ln_body(*refs)
  in_specs = [
      pl.BlockSpec((TOK, HH), lambda sh, b: (b * nsh + sh, 0)),
      pl.BlockSpec((TOK, HH), lambda sh, b: (b * nsh + sh, 1)),
      pl.BlockSpec((TOK, HH), lambda sh, b: (sh, 0)),
      pl.BlockSpec((TOK, HH), lambda sh, b: (sh, 1)),
      pl.BlockSpec((1, 1, TOK), lambda sh, b, base=base, nsh=nsh:
                   ((base + b) * nsh + sh, 0, 0)),
      pl.BlockSpec((1, 1, TOK), lambda sh, b, base=base, nsh=nsh:
                   ((base + b) * nsh + sh, 0, 0)),
      pl.BlockSpec((2, H), lambda sh, b: (0, 0)),
      pl.BlockSpec((8, H), lambda sh, b: (0, 0)),
      pl.BlockSpec((1, H), lambda sh, b: (0, 0)),
      pl.BlockSpec((1, H), lambda sh, b: (0, 0)),
  ]
  args = (gath, gath, pos_used, pos_used, tt_ids, ut_ids, tt_emb, ut_emb,
          gamma2, beta2)
  aliases = {}
  if prev_out is not None:
    in_specs = [pl.BlockSpec(memory_space=pl.ANY)] + in_specs
    args = (prev_out,) + args
    aliases = {0: 0}
  return pl.pallas_call(
      body,
      grid=(nsh, nbh),
      in_specs=in_specs,
      out_specs=pl.BlockSpec((TOK, H), lambda sh, b, base=base, nsh=nsh:
                             ((base + b) * nsh + sh, 0)),
      out_shape=jax.ShapeDtypeStruct((T, H), jnp.float32),
      input_output_aliases=aliases,
  )(*args)


def kernel(input_ids, token_type_ids, user_type_ids, word_emb, pos_emb, tt_emb,
           ut_emb, gamma, beta):
  pos_used = lax.slice(pos_emb, (OFFSET, 0), (OFFSET + S, H))
  tt_ids = token_type_ids.reshape(T // TOK, 1, TOK).astype(jnp.int32)
  ut_ids = user_type_ids.reshape(T // TOK, 1, TOK).astype(jnp.int32)
  gamma2 = gamma.reshape(1, H)
  beta2 = beta.reshape(1, H)
  ids_seg = input_ids.reshape(NSEG, NW * NCHUNK, CHUNK).astype(jnp.int32)

  gaths = [_sc_gather(word_emb, ids_seg[s]) for s in range(NSEG)]
  out = None
  for s in range(NSEG):
    out = _tc_embed_ln(s, out, gaths[s], pos_used, tt_ids, ut_ids,
                       tt_emb, ut_emb, gamma2, beta2)
  return out.reshape(B, S, H)
